# Initial kernel scaffold; baseline (speedup 1.0000x reference)
import jax
import jax.numpy as jnp
from jax.experimental import pallas as pl


def kernel(x, edge_index, edge_weight, W1_l, W1_r, b1, W3_l, W3_r, b3):
    raise NotImplementedError("write your pallas kernel here")


# speedup vs baseline: 17.0004x; 17.0004x over previous
import dataclasses
import functools

import jax
import jax.numpy as jnp
from jax import lax
from jax.experimental import pallas as pl
from jax.experimental.pallas import tpu as pltpu
from jax.experimental.pallas import tpu_sc as plsc

N = 100000
E = 3200000
DIN = 16
D = 8
DP = 16
NC = 2
NS = 16
NW = NC * NS
LANES = 128
KROWS = 8
CHUNK = KROWS * LANES
NCHUNK = E // CHUNK
EROWS = E // LANES
ZROWS = 1000
NZBLK = N // ZROWS


def _sc_aggregate(table, srcm, dstm, wm):
    mesh = plsc.VectorSubcoreMesh(core_axis_name="c", subcore_axis_name="s")
    cp = pltpu.CompilerParams()
    if "needs_layout_passes" in pltpu.CompilerParams.__dataclass_fields__:
        cp = dataclasses.replace(cp, needs_layout_passes=False)
    if "use_tc_tiling_on_sc" in pltpu.CompilerParams.__dataclass_fields__:
        cp = dataclasses.replace(cp, use_tc_tiling_on_sc=False)

    @functools.partial(
        pl.kernel,
        out_type=jax.ShapeDtypeStruct((NC, N, DP), jnp.float32),
        mesh=mesh,
        compiler_params=cp,
        scratch_types=[
            pltpu.VMEM_SHARED((N, DP), jnp.float32),
            pltpu.VMEM((KROWS, LANES), jnp.int32),
            pltpu.VMEM((KROWS, LANES), jnp.int32),
            pltpu.VMEM((KROWS, LANES), jnp.float32),
            pltpu.VMEM((CHUNK, DP), jnp.float32),
            pltpu.SemaphoreType.DMA,
        ],
    )
    def agg_kernel(table_hbm, src_hbm, dst_hbm, w_hbm, out_hbm,
                   acc, src_v, dst_v, w_v, rows_v, sem):
        core = lax.axis_index("c")
        sub = lax.axis_index("s")
        wid = sub * NC + core

        zeros_row = jnp.zeros((DP,), jnp.float32)

        @pl.loop(0, ZROWS)
        def _(i):
            rows_v[i] = zeros_row

        @pl.loop(sub, NZBLK, step=NS)
        def _(b):
            pltpu.sync_copy(rows_v.at[pl.ds(0, ZROWS)],
                            acc.at[pl.ds(b * ZROWS, ZROWS)])

        plsc.subcore_barrier()

        lane = lax.iota(jnp.int32, 16)
        is_feat = lane < D

        @pl.loop(0, (NCHUNK + NW - 1) // NW)
        def _(i):
            cidx = wid + i * NW

            @pl.when(cidx < NCHUNK)
            def _():
                base = cidx * KROWS
                pltpu.sync_copy(src_hbm.at[pl.ds(base, KROWS)], src_v)
                pltpu.sync_copy(dst_hbm.at[pl.ds(base, KROWS)], dst_v)
                pltpu.sync_copy(w_hbm.at[pl.ds(base, KROWS)], w_v)

                gathers = [
                    pltpu.async_copy(
                        table_hbm.at[src_v.at[j]],
                        rows_v.at[pl.ds(j * LANES, LANES)], sem)
                    for j in range(KROWS)
                ]
                for g in gathers:
                    g.wait()

                @pl.loop(0, CHUNK)
                def _(e):
                    j = lax.shift_right_logical(e, 7)
                    l = jnp.bitwise_and(e, 127)
                    w16 = plsc.load_gather(
                        w_v, [jnp.full((16,), j, jnp.int32),
                              jnp.full((16,), l, jnp.int32)])
                    wsel = jnp.where(is_feat, w16, 1.0)
                    rows_v[e] = rows_v[e] * wsel

                scatters = [
                    pltpu.async_copy(
                        rows_v.at[pl.ds(j * LANES, LANES)],
                        acc.at[dst_v.at[j]], sem, add=True)
                    for j in range(KROWS)
                ]
                for s in scatters:
                    s.wait()

        plsc.subcore_barrier()

        @pl.loop(sub, NZBLK, step=NS)
        def _(b):
            pltpu.sync_copy(acc.at[pl.ds(b * ZROWS, ZROWS)],
                            out_hbm.at[core, pl.ds(b * ZROWS, ZROWS)])

    return agg_kernel(table, srcm, dstm, wm)


_BR = 2000


def _tc_pre(x, W_l, W_r, b):
    def body(x_ref, wl_ref, wr_ref, b_ref, t_ref, xr_ref):
        xv = x_ref[...]
        y = lax.dot_general(xv, wl_ref[...], (((1,), (1,)), ((), ())),
                            preferred_element_type=jnp.float32)
        t_ref[...] = jnp.concatenate(
            [y, jnp.ones((_BR, 1), jnp.float32),
             jnp.zeros((_BR, DP - D - 1), jnp.float32)], axis=1)
        xr_ref[...] = lax.dot_general(
            xv, wr_ref[...], (((1,), (1,)), ((), ())),
            preferred_element_type=jnp.float32) + b_ref[...]

    return pl.pallas_call(
        body,
        grid=(N // _BR,),
        in_specs=[
            pl.BlockSpec((_BR, DIN), lambda i: (i, 0)),
            pl.BlockSpec((D, DIN), lambda i: (0, 0)),
            pl.BlockSpec((D, DIN), lambda i: (0, 0)),
            pl.BlockSpec((1, D), lambda i: (0, 0)),
        ],
        out_specs=[
            pl.BlockSpec((_BR, DP), lambda i: (i, 0)),
            pl.BlockSpec((_BR, D), lambda i: (i, 0)),
        ],
        out_shape=[
            jax.ShapeDtypeStruct((N, DP), jnp.float32),
            jax.ShapeDtypeStruct((N, D), jnp.float32),
        ],
    )(x, W_l, W_r, b)


def _tc_mid(P0, P1, xr, W_l, W_r, b):
    def body(p0_ref, p1_ref, xr_ref, wl_ref, wr_ref, b_ref, t_ref, hr_ref):
        s = p0_ref[...] + p1_ref[...]
        deg = jnp.maximum(s[:, D:D + 1], 1.0)
        agg = s[:, :D] / deg
        h = jax.nn.relu(agg + xr_ref[...])
        y = lax.dot_general(h, wl_ref[...], (((1,), (1,)), ((), ())),
                            preferred_element_type=jnp.float32)
        t_ref[...] = jnp.concatenate(
            [y, jnp.ones((_BR, 1), jnp.float32),
             jnp.zeros((_BR, DP - D - 1), jnp.float32)], axis=1)
        hr_ref[...] = lax.dot_general(
            h, wr_ref[...], (((1,), (1,)), ((), ())),
            preferred_element_type=jnp.float32) + b_ref[...]

    return pl.pallas_call(
        body,
        grid=(N // _BR,),
        in_specs=[
            pl.BlockSpec((_BR, DP), lambda i: (i, 0)),
            pl.BlockSpec((_BR, DP), lambda i: (i, 0)),
            pl.BlockSpec((_BR, D), lambda i: (i, 0)),
            pl.BlockSpec((D, D), lambda i: (0, 0)),
            pl.BlockSpec((D, D), lambda i: (0, 0)),
            pl.BlockSpec((1, D), lambda i: (0, 0)),
        ],
        out_specs=[
            pl.BlockSpec((_BR, DP), lambda i: (i, 0)),
            pl.BlockSpec((_BR, D), lambda i: (i, 0)),
        ],
        out_shape=[
            jax.ShapeDtypeStruct((N, DP), jnp.float32),
            jax.ShapeDtypeStruct((N, D), jnp.float32),
        ],
    )(P0, P1, xr, W_l, W_r, b)


def _tc_post(Q0, Q1, hr):
    def body(q0_ref, q1_ref, hr_ref, o_ref):
        s = q0_ref[...] + q1_ref[...]
        deg = jnp.maximum(s[:, D:D + 1], 1.0)
        o = s[:, :D] / deg + hr_ref[...]
        z = o - jnp.max(o, axis=1, keepdims=True)
        lse = jnp.log(jnp.sum(jnp.exp(z), axis=1, keepdims=True))
        o_ref[...] = z - lse

    return pl.pallas_call(
        body,
        grid=(N // _BR,),
        in_specs=[
            pl.BlockSpec((_BR, DP), lambda i: (i, 0)),
            pl.BlockSpec((_BR, DP), lambda i: (i, 0)),
            pl.BlockSpec((_BR, D), lambda i: (i, 0)),
        ],
        out_specs=pl.BlockSpec((_BR, D), lambda i: (i, 0)),
        out_shape=jax.ShapeDtypeStruct((N, D), jnp.float32),
    )(Q0, Q1, hr)


def kernel(x, edge_index, edge_weight, W1_l, W1_r, b1, W3_l, W3_r, b3):
    src = edge_index[0].reshape(EROWS, LANES)
    dst = edge_index[1].reshape(EROWS, LANES)
    w = edge_weight.reshape(EROWS, LANES)
    b1r = jnp.reshape(b1, (1, D))
    b3r = jnp.reshape(b3, (1, D))

    table1, xr = _tc_pre(x, W1_l, W1_r, b1r)
    P = _sc_aggregate(table1, src, dst, w)
    table2, hr = _tc_mid(P[0], P[1], xr, W3_l, W3_r, b3r)
    Q = _sc_aggregate(table2, src, dst, w)
    return _tc_post(Q[0], Q[1], hr)


# speedup vs baseline: 24.5861x; 1.4462x over previous
import dataclasses
import functools

import jax
import jax.numpy as jnp
from jax import lax
from jax.experimental import pallas as pl
from jax.experimental.pallas import tpu as pltpu
from jax.experimental.pallas import tpu_sc as plsc

N = 100000
E = 3200000
DIN = 16
D = 8
NC = 2
NS = 16
NW = NC * NS
LANES = 128
KROWS = 8
CHUNK = KROWS * LANES
NCHUNK = E // CHUNK
EROWS = E // LANES
ZROWS = 1000
NZBLK = N // ZROWS


def _sc_params():
    cp = pltpu.CompilerParams()
    if "needs_layout_passes" in pltpu.CompilerParams.__dataclass_fields__:
        cp = dataclasses.replace(cp, needs_layout_passes=False)
    if "use_tc_tiling_on_sc" in pltpu.CompilerParams.__dataclass_fields__:
        cp = dataclasses.replace(cp, use_tc_tiling_on_sc=False)
    return cp


def _sc_mesh():
    return plsc.VectorSubcoreMesh(core_axis_name="c", subcore_axis_name="s")


def _sc_aggregate(table, srcm, dstm, wvec, zblk):
    @functools.partial(
        pl.kernel,
        out_type=jax.ShapeDtypeStruct((NC, N, D), jnp.float32),
        mesh=_sc_mesh(),
        compiler_params=_sc_params(),
        scratch_types=[
            pltpu.VMEM_SHARED((N, D), jnp.float32),
            pltpu.VMEM((KROWS, LANES), jnp.int32),
            pltpu.VMEM((KROWS, LANES), jnp.int32),
            pltpu.VMEM((CHUNK,), jnp.float32),
            pltpu.VMEM((CHUNK, D), jnp.float32),
            pltpu.SemaphoreType.DMA,
        ],
    )
    def agg_kernel(table_hbm, src_hbm, dst_hbm, w_hbm, z_hbm, out_hbm,
                   acc, src_v, dst_v, w_v, rows_v, sem):
        core = lax.axis_index("c")
        sub = lax.axis_index("s")
        wid = sub * NC + core

        @pl.loop(sub, NZBLK, step=NS)
        def _(b):
            pltpu.sync_copy(z_hbm, acc.at[pl.ds(b * ZROWS, ZROWS)])

        plsc.subcore_barrier()

        half = jnp.where(lax.iota(jnp.int32, 16) < D, 0, 1)
        colc = jnp.bitwise_and(lax.iota(jnp.int32, 16), D - 1)

        @pl.loop(0, (NCHUNK + NW - 1) // NW)
        def _(i):
            cidx = wid + i * NW

            @pl.when(cidx < NCHUNK)
            def _():
                base = cidx * KROWS
                pltpu.sync_copy(src_hbm.at[pl.ds(base, KROWS)], src_v)
                pltpu.sync_copy(dst_hbm.at[pl.ds(base, KROWS)], dst_v)
                pltpu.sync_copy(w_hbm.at[pl.ds(cidx * CHUNK, CHUNK)], w_v)

                gathers = [
                    pltpu.async_copy(table_hbm.at[src_v.at[j]],
                                     rows_v.at[pl.ds(j * LANES, LANES)], sem)
                    for j in range(KROWS)
                ]
                for g in gathers:
                    g.wait()

                @plsc.parallel_loop(0, CHUNK // 2, unroll=4)
                def _(p):
                    ridx = jnp.full((16,), 2 * p, jnp.int32) + half
                    w16 = plsc.load_gather(w_v, [ridx])
                    r16 = plsc.load_gather(rows_v, [ridx, colc])
                    plsc.store_scatter(rows_v, [ridx, colc], r16 * w16)

                scatters = [
                    pltpu.async_copy(rows_v.at[pl.ds(j * LANES, LANES)],
                                     acc.at[dst_v.at[j]], sem, add=True)
                    for j in range(KROWS)
                ]
                for s in scatters:
                    s.wait()

        plsc.subcore_barrier()

        @pl.loop(sub, NZBLK, step=NS)
        def _(b):
            pltpu.sync_copy(acc.at[pl.ds(b * ZROWS, ZROWS)],
                            out_hbm.at[core, pl.ds(b * ZROWS, ZROWS)])

    return agg_kernel(table, srcm, dstm, wvec, zblk)


def _sc_degree(dstm, ones_blk, zblk):
    @functools.partial(
        pl.kernel,
        out_type=jax.ShapeDtypeStruct((NC, N, D), jnp.float32),
        mesh=_sc_mesh(),
        compiler_params=_sc_params(),
        scratch_types=[
            pltpu.VMEM_SHARED((N, D), jnp.float32),
            pltpu.VMEM((KROWS, LANES), jnp.int32),
            pltpu.VMEM((LANES, D), jnp.float32),
            pltpu.SemaphoreType.DMA,
        ],
    )
    def deg_kernel(dst_hbm, ones_hbm, z_hbm, out_hbm, acc, dst_v, ones_v, sem):
        core = lax.axis_index("c")
        sub = lax.axis_index("s")
        wid = sub * NC + core

        pltpu.sync_copy(ones_hbm, ones_v)

        @pl.loop(sub, NZBLK, step=NS)
        def _(b):
            pltpu.sync_copy(z_hbm, acc.at[pl.ds(b * ZROWS, ZROWS)])

        plsc.subcore_barrier()

        @pl.loop(0, (NCHUNK + NW - 1) // NW)
        def _(i):
            cidx = wid + i * NW

            @pl.when(cidx < NCHUNK)
            def _():
                pltpu.sync_copy(dst_hbm.at[pl.ds(cidx * KROWS, KROWS)], dst_v)
                scatters = [
                    pltpu.async_copy(ones_v, acc.at[dst_v.at[j]], sem,
                                     add=True)
                    for j in range(KROWS)
                ]
                for s in scatters:
                    s.wait()

        plsc.subcore_barrier()

        @pl.loop(sub, NZBLK, step=NS)
        def _(b):
            pltpu.sync_copy(acc.at[pl.ds(b * ZROWS, ZROWS)],
                            out_hbm.at[core, pl.ds(b * ZROWS, ZROWS)])

    return deg_kernel(dstm, ones_blk, zblk)


_BR = 2000


def _tc_pre(x, W_l, W_r, b):
    def body(x_ref, wl_ref, wr_ref, b_ref, t_ref, xr_ref):
        xv = x_ref[...]
        t_ref[...] = lax.dot_general(xv, wl_ref[...], (((1,), (1,)), ((), ())),
                                     preferred_element_type=jnp.float32)
        xr_ref[...] = lax.dot_general(
            xv, wr_ref[...], (((1,), (1,)), ((), ())),
            preferred_element_type=jnp.float32) + b_ref[...]

    return pl.pallas_call(
        body,
        grid=(N // _BR,),
        in_specs=[
            pl.BlockSpec((_BR, DIN), lambda i: (i, 0)),
            pl.BlockSpec((D, DIN), lambda i: (0, 0)),
            pl.BlockSpec((D, DIN), lambda i: (0, 0)),
            pl.BlockSpec((1, D), lambda i: (0, 0)),
        ],
        out_specs=[
            pl.BlockSpec((_BR, D), lambda i: (i, 0)),
            pl.BlockSpec((_BR, D), lambda i: (i, 0)),
        ],
        out_shape=[
            jax.ShapeDtypeStruct((N, D), jnp.float32),
            jax.ShapeDtypeStruct((N, D), jnp.float32),
        ],
    )(x, W_l, W_r, b)


def _tc_mid(P0, P1, G0, G1, xr, W_l, W_r, b):
    def body(p0_ref, p1_ref, g0_ref, g1_ref, xr_ref, wl_ref, wr_ref, b_ref,
             t_ref, hr_ref):
        deg = jnp.maximum(g0_ref[...] + g1_ref[...], 1.0)
        h = jax.nn.relu((p0_ref[...] + p1_ref[...]) / deg + xr_ref[...])
        t_ref[...] = lax.dot_general(h, wl_ref[...], (((1,), (1,)), ((), ())),
                                     preferred_element_type=jnp.float32)
        hr_ref[...] = lax.dot_general(
            h, wr_ref[...], (((1,), (1,)), ((), ())),
            preferred_element_type=jnp.float32) + b_ref[...]

    return pl.pallas_call(
        body,
        grid=(N // _BR,),
        in_specs=[
            pl.BlockSpec((_BR, D), lambda i: (i, 0)),
            pl.BlockSpec((_BR, D), lambda i: (i, 0)),
            pl.BlockSpec((_BR, D), lambda i: (i, 0)),
            pl.BlockSpec((_BR, D), lambda i: (i, 0)),
            pl.BlockSpec((_BR, D), lambda i: (i, 0)),
            pl.BlockSpec((D, D), lambda i: (0, 0)),
            pl.BlockSpec((D, D), lambda i: (0, 0)),
            pl.BlockSpec((1, D), lambda i: (0, 0)),
        ],
        out_specs=[
            pl.BlockSpec((_BR, D), lambda i: (i, 0)),
            pl.BlockSpec((_BR, D), lambda i: (i, 0)),
        ],
        out_shape=[
            jax.ShapeDtypeStruct((N, D), jnp.float32),
            jax.ShapeDtypeStruct((N, D), jnp.float32),
        ],
    )(P0, P1, G0, G1, xr, W_l, W_r, b)


def _tc_post(Q0, Q1, G0, G1, hr):
    def body(q0_ref, q1_ref, g0_ref, g1_ref, hr_ref, o_ref):
        deg = jnp.maximum(g0_ref[...] + g1_ref[...], 1.0)
        o = (q0_ref[...] + q1_ref[...]) / deg + hr_ref[...]
        z = o - jnp.max(o, axis=1, keepdims=True)
        lse = jnp.log(jnp.sum(jnp.exp(z), axis=1, keepdims=True))
        o_ref[...] = z - lse

    return pl.pallas_call(
        body,
        grid=(N // _BR,),
        in_specs=[
            pl.BlockSpec((_BR, D), lambda i: (i, 0)),
            pl.BlockSpec((_BR, D), lambda i: (i, 0)),
            pl.BlockSpec((_BR, D), lambda i: (i, 0)),
            pl.BlockSpec((_BR, D), lambda i: (i, 0)),
            pl.BlockSpec((_BR, D), lambda i: (i, 0)),
        ],
        out_specs=pl.BlockSpec((_BR, D), lambda i: (i, 0)),
        out_shape=jax.ShapeDtypeStruct((N, D), jnp.float32),
    )(Q0, Q1, G0, G1, hr)


def kernel(x, edge_index, edge_weight, W1_l, W1_r, b1, W3_l, W3_r, b3):
    src = edge_index[0].reshape(EROWS, LANES)
    dst = edge_index[1].reshape(EROWS, LANES)
    zblk = jnp.zeros((ZROWS, D), jnp.float32)
    ones_blk = jnp.ones((LANES, D), jnp.float32)
    b1r = jnp.reshape(b1, (1, D))
    b3r = jnp.reshape(b3, (1, D))

    G = _sc_degree(dst, ones_blk, zblk)
    table1, xr = _tc_pre(x, W1_l, W1_r, b1r)
    P = _sc_aggregate(table1, src, dst, edge_weight, zblk)
    table2, hr = _tc_mid(P[0], P[1], G[0], G[1], xr, W3_l, W3_r, b3r)
    Q = _sc_aggregate(table2, src, dst, edge_weight, zblk)
    return _tc_post(Q[0], Q[1], G[0], G[1], hr)


# speedup vs baseline: 32.1620x; 1.3081x over previous
import dataclasses
import functools

import jax
import jax.numpy as jnp
from jax import lax
from jax.experimental import pallas as pl
from jax.experimental.pallas import tpu as pltpu
from jax.experimental.pallas import tpu_sc as plsc

N = 100000
E = 3200000
DIN = 16
D = 8
NC = 2
NS = 16
NW = NC * NS
LANES = 128
KROWS = 8
CHUNK = KROWS * LANES
NCHUNK = E // CHUNK
SUP = 8
SEDGES = SUP * CHUNK
NSUP = (NCHUNK + SUP - 1) // SUP
EPAD = NSUP * SEDGES
EROWS = E // LANES
ERPAD = EPAD // LANES
ZROWS = 1000
NZBLK = N // ZROWS


def _sc_params():
    cp = pltpu.CompilerParams()
    if "needs_layout_passes" in pltpu.CompilerParams.__dataclass_fields__:
        cp = dataclasses.replace(cp, needs_layout_passes=False)
    if "use_tc_tiling_on_sc" in pltpu.CompilerParams.__dataclass_fields__:
        cp = dataclasses.replace(cp, use_tc_tiling_on_sc=False)
    return cp


def _sc_mesh():
    return plsc.VectorSubcoreMesh(core_axis_name="c", subcore_axis_name="s")


def _sc_aggregate(table, src1d, dstm, wvec, zblk):
    @functools.partial(
        pl.kernel,
        out_type=jax.ShapeDtypeStruct((NC, N, D), jnp.float32),
        mesh=_sc_mesh(),
        compiler_params=_sc_params(),
        scratch_types=[
            pltpu.VMEM_SHARED((N + 8, D), jnp.float32),
            pltpu.VMEM((SEDGES,), jnp.int32),
            pltpu.VMEM((SEDGES // LANES, LANES), jnp.int32),
            pltpu.VMEM((SEDGES,), jnp.float32),
            pltpu.VMEM((CHUNK, D), jnp.float32),
            pltpu.VMEM((CHUNK, D), jnp.float32),
            pltpu.SemaphoreType.DMA,
            pltpu.SemaphoreType.DMA,
            pltpu.SemaphoreType.DMA,
            pltpu.SemaphoreType.DMA,
        ],
    )
    def agg_kernel(table_hbm, src_hbm, dst_hbm, w_hbm, z_hbm, out_hbm,
                   acc, src_v, dst_v, w_v, rows_a, rows_b,
                   gsem0, gsem1, ssem0, ssem1):
        core = lax.axis_index("c")
        sub = lax.axis_index("s")
        wid = sub * NC + core
        rows = (rows_a, rows_b)
        gsem = (gsem0, gsem1)
        ssem = (ssem0, ssem1)

        @pl.loop(sub, NZBLK, step=NS)
        def _(b):
            pltpu.sync_copy(z_hbm, acc.at[pl.ds(b * ZROWS, ZROWS)])

        plsc.subcore_barrier()

        half = jnp.where(lax.iota(jnp.int32, 16) < D, 0, 1)
        colc = jnp.bitwise_and(lax.iota(jnp.int32, 16), D - 1)

        @pl.loop(0, (NSUP + NW - 1) // NW)
        def _(i):
            s = wid + i * NW

            @pl.when(s < NSUP)
            def _():
                ebase = s * SEDGES
                pltpu.sync_copy(src_hbm.at[pl.ds(ebase, SEDGES)], src_v)
                pltpu.sync_copy(
                    dst_hbm.at[pl.ds(s * (SEDGES // LANES), SEDGES // LANES)],
                    dst_v)
                pltpu.sync_copy(w_hbm.at[pl.ds(ebase, SEDGES)], w_v)

                def start_gather(k):
                    return pltpu.async_copy(
                        table_hbm.at[src_v.at[pl.ds(k * CHUNK, CHUNK)]],
                        rows[k % 2], gsem[k % 2])

                def start_scatter(k):
                    return [
                        pltpu.async_copy(
                            rows[k % 2].at[pl.ds(j * LANES, LANES)],
                            acc.at[dst_v.at[k * KROWS + j]],
                            ssem[k % 2], add=True)
                        for j in range(KROWS)
                    ]

                def compute(k):
                    rv = rows[k % 2]
                    wbase = k * CHUNK

                    @plsc.parallel_loop(0, CHUNK // 2, unroll=4)
                    def _(p):
                        e2 = 2 * p
                        ridx = jnp.full((16,), e2, jnp.int32) + half
                        w16 = plsc.load_gather(
                            w_v, [jnp.full((16,), wbase + e2, jnp.int32)
                                  + half])
                        r16 = plsc.load_gather(rv, [ridx, colc])
                        plsc.store_scatter(rv, [ridx, colc], r16 * w16)

                gd = [None] * SUP
                sd = [None] * SUP
                gd[0] = start_gather(0)
                for k in range(SUP):
                    if k + 1 < SUP:
                        if k >= 1:
                            for d in sd[k - 1]:
                                d.wait()
                        gd[k + 1] = start_gather(k + 1)
                    gd[k].wait()
                    compute(k)
                    sd[k] = start_scatter(k)
                for d in sd[SUP - 2] + sd[SUP - 1]:
                    d.wait()

        plsc.subcore_barrier()

        @pl.loop(sub, NZBLK, step=NS)
        def _(b):
            pltpu.sync_copy(acc.at[pl.ds(b * ZROWS, ZROWS)],
                            out_hbm.at[core, pl.ds(b * ZROWS, ZROWS)])

    return agg_kernel(table, src1d, dstm, wvec, zblk)


def _sc_degree(dstm, ones_blk, zblk):
    SROWS = SEDGES // LANES

    @functools.partial(
        pl.kernel,
        out_type=jax.ShapeDtypeStruct((NC, N, D), jnp.float32),
        mesh=_sc_mesh(),
        compiler_params=_sc_params(),
        scratch_types=[
            pltpu.VMEM_SHARED((N + 8, D), jnp.float32),
            pltpu.VMEM((SROWS, LANES), jnp.int32),
            pltpu.VMEM((LANES, D), jnp.float32),
            pltpu.SemaphoreType.DMA,
        ],
    )
    def deg_kernel(dst_hbm, ones_hbm, z_hbm, out_hbm, acc, dst_v, ones_v, sem):
        core = lax.axis_index("c")
        sub = lax.axis_index("s")
        wid = sub * NC + core

        pltpu.sync_copy(ones_hbm, ones_v)

        @pl.loop(sub, NZBLK, step=NS)
        def _(b):
            pltpu.sync_copy(z_hbm, acc.at[pl.ds(b * ZROWS, ZROWS)])

        plsc.subcore_barrier()

        @pl.loop(0, (NSUP + NW - 1) // NW)
        def _(i):
            s = wid + i * NW

            @pl.when(s < NSUP)
            def _():
                pltpu.sync_copy(dst_hbm.at[pl.ds(s * SROWS, SROWS)], dst_v)
                scatters = [
                    pltpu.async_copy(ones_v, acc.at[dst_v.at[j]], sem,
                                     add=True)
                    for j in range(SROWS)
                ]
                for d in scatters:
                    d.wait()

        plsc.subcore_barrier()

        @pl.loop(sub, NZBLK, step=NS)
        def _(b):
            pltpu.sync_copy(acc.at[pl.ds(b * ZROWS, ZROWS)],
                            out_hbm.at[core, pl.ds(b * ZROWS, ZROWS)])

    return deg_kernel(dstm, ones_blk, zblk)


_BR = 2000


def _tc_pre(x, W_l, W_r, b):
    def body(x_ref, wl_ref, wr_ref, b_ref, t_ref, xr_ref):
        xv = x_ref[...]
        t_ref[...] = lax.dot_general(xv, wl_ref[...], (((1,), (1,)), ((), ())),
                                     preferred_element_type=jnp.float32)
        xr_ref[...] = lax.dot_general(
            xv, wr_ref[...], (((1,), (1,)), ((), ())),
            preferred_element_type=jnp.float32) + b_ref[...]

    return pl.pallas_call(
        body,
        grid=(N // _BR,),
        in_specs=[
            pl.BlockSpec((_BR, DIN), lambda i: (i, 0)),
            pl.BlockSpec((D, DIN), lambda i: (0, 0)),
            pl.BlockSpec((D, DIN), lambda i: (0, 0)),
            pl.BlockSpec((1, D), lambda i: (0, 0)),
        ],
        out_specs=[
            pl.BlockSpec((_BR, D), lambda i: (i, 0)),
            pl.BlockSpec((_BR, D), lambda i: (i, 0)),
        ],
        out_shape=[
            jax.ShapeDtypeStruct((N, D), jnp.float32),
            jax.ShapeDtypeStruct((N, D), jnp.float32),
        ],
    )(x, W_l, W_r, b)


def _tc_mid(P0, P1, G0, G1, xr, W_l, W_r, b):
    def body(p0_ref, p1_ref, g0_ref, g1_ref, xr_ref, wl_ref, wr_ref, b_ref,
             t_ref, hr_ref):
        deg = jnp.maximum(g0_ref[...] + g1_ref[...], 1.0)
        h = jax.nn.relu((p0_ref[...] + p1_ref[...]) / deg + xr_ref[...])
        t_ref[...] = lax.dot_general(h, wl_ref[...], (((1,), (1,)), ((), ())),
                                     preferred_element_type=jnp.float32)
        hr_ref[...] = lax.dot_general(
            h, wr_ref[...], (((1,), (1,)), ((), ())),
            preferred_element_type=jnp.float32) + b_ref[...]

    return pl.pallas_call(
        body,
        grid=(N // _BR,),
        in_specs=[
            pl.BlockSpec((_BR, D), lambda i: (i, 0)),
            pl.BlockSpec((_BR, D), lambda i: (i, 0)),
            pl.BlockSpec((_BR, D), lambda i: (i, 0)),
            pl.BlockSpec((_BR, D), lambda i: (i, 0)),
            pl.BlockSpec((_BR, D), lambda i: (i, 0)),
            pl.BlockSpec((D, D), lambda i: (0, 0)),
            pl.BlockSpec((D, D), lambda i: (0, 0)),
            pl.BlockSpec((1, D), lambda i: (0, 0)),
        ],
        out_specs=[
            pl.BlockSpec((_BR, D), lambda i: (i, 0)),
            pl.BlockSpec((_BR, D), lambda i: (i, 0)),
        ],
        out_shape=[
            jax.ShapeDtypeStruct((N, D), jnp.float32),
            jax.ShapeDtypeStruct((N, D), jnp.float32),
        ],
    )(P0, P1, G0, G1, xr, W_l, W_r, b)


def _tc_post(Q0, Q1, G0, G1, hr):
    def body(q0_ref, q1_ref, g0_ref, g1_ref, hr_ref, o_ref):
        deg = jnp.maximum(g0_ref[...] + g1_ref[...], 1.0)
        o = (q0_ref[...] + q1_ref[...]) / deg + hr_ref[...]
        z = o - jnp.max(o, axis=1, keepdims=True)
        lse = jnp.log(jnp.sum(jnp.exp(z), axis=1, keepdims=True))
        o_ref[...] = z - lse

    return pl.pallas_call(
        body,
        grid=(N // _BR,),
        in_specs=[
            pl.BlockSpec((_BR, D), lambda i: (i, 0)),
            pl.BlockSpec((_BR, D), lambda i: (i, 0)),
            pl.BlockSpec((_BR, D), lambda i: (i, 0)),
            pl.BlockSpec((_BR, D), lambda i: (i, 0)),
            pl.BlockSpec((_BR, D), lambda i: (i, 0)),
        ],
        out_specs=pl.BlockSpec((_BR, D), lambda i: (i, 0)),
        out_shape=jax.ShapeDtypeStruct((N, D), jnp.float32),
    )(Q0, Q1, G0, G1, hr)


def kernel(x, edge_index, edge_weight, W1_l, W1_r, b1, W3_l, W3_r, b3):
    pad = EPAD - E
    src1d = jnp.pad(edge_index[0], (0, pad))
    dst = jnp.pad(edge_index[1], (0, pad),
                  constant_values=N).reshape(ERPAD, LANES)
    wpad = jnp.pad(edge_weight, (0, pad))
    zblk = jnp.zeros((ZROWS, D), jnp.float32)
    ones_blk = jnp.ones((LANES, D), jnp.float32)
    b1r = jnp.reshape(b1, (1, D))
    b3r = jnp.reshape(b3, (1, D))

    G = _sc_degree(dst, ones_blk, zblk)
    table1, xr = _tc_pre(x, W1_l, W1_r, b1r)
    P = _sc_aggregate(table1, src1d, dst, wpad, zblk)
    table2, hr = _tc_mid(P[0], P[1], G[0], G[1], xr, W3_l, W3_r, b3r)
    Q = _sc_aggregate(table2, src1d, dst, wpad, zblk)
    return _tc_post(Q[0], Q[1], G[0], G[1], hr)
